# Initial kernel scaffold; baseline (speedup 1.0000x reference)
import jax
import jax.numpy as jnp
from jax.experimental import pallas as pl


def kernel(fake_img_s, fake_img_t):
    raise NotImplementedError("write your pallas kernel here")


# speedup vs baseline: 3.4872x; 3.4872x over previous
import functools

import jax
import jax.numpy as jnp
from jax import lax
from jax.experimental import pallas as pl
from jax.experimental.pallas import tpu as pltpu
from jax.experimental.pallas import tpu_sc as plsc

BINS = 256
NC = 2
NS = 16
NW = NC * NS
LANES = 16

B, C, H, W = 4, 3, 224, 224
PIX = H * W
NBLK = B * C
CHUNK = PIX // NW
VECS = CHUNK // LANES
NCH = 2 * C
ACC_W = NCH * BINS


def _sc_hist(s_hbm, t_hbm, out_hbm, buf, acc, part, sem):
    wid = lax.axis_index("s") * NC + lax.axis_index("c")

    zeros = jnp.zeros((LANES,), jnp.float32)

    def zero_body(j, _):
        acc[pl.ds(j * LANES, LANES)] = zeros
        return _

    lax.fori_loop(0, (LANES * ACC_W) // LANES, zero_body, None)

    copies = []
    for k in range(2 * NBLK):
        src = s_hbm if k < NBLK else t_hbm
        blk = k % NBLK
        start = blk * PIX
        copies.append(
            pltpu.async_copy(
                src.at[pl.ds(start + wid * CHUNK, CHUNK)],
                buf.at[pl.ds(k * CHUNK, CHUNK)],
                sem,
            )
        )
    for cp in copies:
        cp.wait()

    lane_off = lax.iota(jnp.int32, LANES) * ACC_W

    for k in range(2 * NBLK):
        img = 0 if k < NBLK else 1
        ch6 = img * C + (k % C)
        choff = ch6 * BINS
        base = k * CHUNK

        def body(i, _, base=base, choff=choff):
            v = buf[pl.ds(base + i * LANES, LANES)]
            x = v * jnp.float32(BINS) - jnp.float32(0.5)
            i0 = (x + jnp.float32(1.0)).astype(jnp.int32) - 1
            f = x - i0.astype(jnp.float32)
            i1 = i0 + 1
            i0c = jnp.clip(i0, 0, BINS - 1) + choff + lane_off
            i1c = jnp.clip(i1, 0, BINS - 1) + choff + lane_off
            m0 = jnp.logical_and(i0 >= 0, i0 <= BINS - 1)
            m1 = jnp.logical_and(i1 >= 0, i1 <= BINS - 1)
            plsc.addupdate_scatter(acc, [i0c], jnp.float32(1.0) - f, mask=m0)
            plsc.addupdate_scatter(acc, [i1c], f, mask=m1)
            return _

        lax.fori_loop(0, VECS, body, None)

    def red_body(cidx, _):
        start = cidx * LANES
        s = acc[pl.ds(start, LANES)]
        for r in range(1, LANES):
            s = s + acc[pl.ds(r * ACC_W + start, LANES)]
        part[pl.ds(start, LANES)] = s
        return _

    lax.fori_loop(0, ACC_W // LANES, red_body, None)

    pltpu.sync_copy(part, out_hbm.at[wid])


def _tc_loss(p_ref, o_ref):
    p = p_ref[...]
    h = jnp.sum(p, axis=0)
    tot = jnp.sum(h, axis=2, keepdims=True)
    r = h / (tot + jnp.float32(1e-10)) + jnp.float32(1e-16)
    a = r[0]
    b = r[1]
    o_ref[0, 0] = jnp.float32(2.0) * jnp.sum((a - b) ** 2 / (a + b))


def kernel(fake_img_s, fake_img_t):
    s_flat = fake_img_s.reshape(-1)
    t_flat = fake_img_t.reshape(-1)

    mesh = plsc.VectorSubcoreMesh(
        core_axis_name="c", subcore_axis_name="s", num_cores=NC, num_subcores=NS
    )
    partials = pl.kernel(
        _sc_hist,
        out_type=jax.ShapeDtypeStruct((NW, ACC_W), jnp.float32),
        mesh=mesh,
        compiler_params=pltpu.CompilerParams(needs_layout_passes=False),
        scratch_types=[
            pltpu.VMEM((2 * NBLK * CHUNK,), jnp.float32),
            pltpu.VMEM((LANES * ACC_W,), jnp.float32),
            pltpu.VMEM((ACC_W,), jnp.float32),
            pltpu.SemaphoreType.DMA,
        ],
    )(s_flat, t_flat)

    loss = pl.pallas_call(
        _tc_loss,
        out_shape=jax.ShapeDtypeStruct((1, 1), jnp.float32),
        out_specs=pl.BlockSpec(memory_space=pltpu.SMEM),
    )(partials.reshape(NW, 2, C, BINS))
    return loss[0, 0]


# speedup vs baseline: 3.8723x; 1.1104x over previous
import jax
import jax.numpy as jnp
from jax import lax
from jax.experimental import pallas as pl
from jax.experimental.pallas import tpu as pltpu
from jax.experimental.pallas import tpu_sc as plsc

BINS = 256
NC = 2
NS = 16
NW = NC * NS
LANES = 16

B, C, H, W = 4, 3, 224, 224
PIX = H * W
NBLK = B * C
CHUNK = PIX // NW
VECS = CHUNK // LANES
UNROLL = 7
NCH = 2 * C
CH_W = 272
ROW_W = NCH * CH_W
ACC_VECS = ROW_W // LANES


def _sc_hist(s_hbm, t_hbm, out_hbm, buf, acc, part, sem):
    wid = lax.axis_index("s") * NC + lax.axis_index("c")

    copies = []
    for k in range(2 * NBLK):
        src = s_hbm if k < NBLK else t_hbm
        start = (k % NBLK) * PIX
        copies.append(
            pltpu.async_copy(
                src.at[pl.ds(start + wid * CHUNK, CHUNK)],
                buf.at[pl.ds(k * CHUNK, CHUNK)],
                sem,
            )
        )

    zeros = jnp.zeros((LANES,), jnp.float32)

    def zero_body(j, _):
        for u in range(4):
            acc[pl.ds((j * 4 + u) * LANES, LANES)] = zeros
        return _

    lax.fori_loop(0, (LANES * ACC_VECS) // 4, zero_body, None)

    for cp in copies:
        cp.wait()

    lane_base = lax.iota(jnp.int32, LANES) * ROW_W

    for k in range(2 * NBLK):
        img = 0 if k < NBLK else 1
        ch6 = img * C + (k % C)
        bidx = lane_base + ch6 * CH_W
        base = k * CHUNK

        def body(i, _, base=base, bidx=bidx):
            for u in range(UNROLL):
                v = buf[pl.ds(base + (i * UNROLL + u) * LANES, LANES)]
                y = v * jnp.float32(BINS) + jnp.float32(0.5)
                y = jnp.minimum(jnp.maximum(y, jnp.float32(0.0)),
                                jnp.float32(BINS + 0.49))
                ji = y.astype(jnp.int32)
                f = y - ji.astype(jnp.float32)
                idx0 = bidx + ji
                plsc.addupdate_scatter(acc, [idx0], jnp.float32(1.0) - f)
                plsc.addupdate_scatter(acc, [idx0 + 1], f)
            return _

        lax.fori_loop(0, VECS // UNROLL, body, None)

    def red_body(cidx, _):
        start = cidx * LANES
        s = acc[pl.ds(start, LANES)]
        for r in range(1, LANES):
            s = s + acc[pl.ds(r * ROW_W + start, LANES)]
        part[pl.ds(start, LANES)] = s
        return _

    lax.fori_loop(0, ACC_VECS, red_body, None)

    pltpu.sync_copy(part, out_hbm.at[wid])


def _tc_loss(p_ref, o_ref):
    p = p_ref[...]
    h = jnp.sum(p, axis=0)
    slot = lax.broadcasted_iota(jnp.int32, h.shape, 2)
    inb = jnp.logical_and(slot >= 1, slot <= BINS)
    h = jnp.where(inb, h, jnp.float32(0.0))
    tot = jnp.sum(h, axis=2, keepdims=True)
    r = h / (tot + jnp.float32(1e-10))
    r = jnp.where(inb, r + jnp.float32(1e-16), jnp.float32(1e-16))
    a = r[0]
    b = r[1]
    o_ref[0, 0] = jnp.float32(2.0) * jnp.sum((a - b) ** 2 / (a + b))


def kernel(fake_img_s, fake_img_t):
    s_flat = fake_img_s.reshape(-1)
    t_flat = fake_img_t.reshape(-1)

    mesh = plsc.VectorSubcoreMesh(
        core_axis_name="c", subcore_axis_name="s", num_cores=NC, num_subcores=NS
    )
    partials = pl.kernel(
        _sc_hist,
        out_type=jax.ShapeDtypeStruct((NW, ROW_W), jnp.float32),
        mesh=mesh,
        compiler_params=pltpu.CompilerParams(needs_layout_passes=False),
        scratch_types=[
            pltpu.VMEM((2 * NBLK * CHUNK,), jnp.float32),
            pltpu.VMEM((LANES * ROW_W,), jnp.float32),
            pltpu.VMEM((ROW_W,), jnp.float32),
            pltpu.SemaphoreType.DMA,
        ],
    )(s_flat, t_flat)

    loss = pl.pallas_call(
        _tc_loss,
        out_shape=jax.ShapeDtypeStruct((1, 1), jnp.float32),
        out_specs=pl.BlockSpec(memory_space=pltpu.SMEM),
    )(partials.reshape(NW, 2, C, CH_W))
    return loss[0, 0]


# speedup vs baseline: 6.8334x; 1.7647x over previous
import jax
import jax.numpy as jnp
from jax import lax
from jax.experimental import pallas as pl
from jax.experimental.pallas import tpu as pltpu
from jax.experimental.pallas import tpu_sc as plsc

BINS = 256
NC = 2
NS = 16
NW = NC * NS
LANES = 16

B, C, H, W = 4, 3, 224, 224
PIX = H * W
NBLK = B * C
CHUNK = PIX // NW
VECS = CHUNK // LANES
UNROLL = 7
NCH = 2 * C
CH_W = 272
ROW_W = NCH * CH_W
ACC_VECS = ROW_W // LANES


def _sc_hist(s_hbm, t_hbm, out_hbm, buf, acc, part, sem):
    wid = lax.axis_index("s") * NC + lax.axis_index("c")

    copies = []
    for k in range(2 * NBLK):
        src = s_hbm if k < NBLK else t_hbm
        start = (k % NBLK) * PIX
        copies.append(
            pltpu.async_copy(
                src.at[pl.ds(start + wid * CHUNK, CHUNK)],
                buf.at[pl.ds(k * CHUNK, CHUNK)],
                sem,
            )
        )

    zeros = jnp.zeros((LANES,), jnp.float32)

    @plsc.parallel_loop(0, LANES * ACC_VECS, unroll=8)
    def zero_body(j):
        acc[pl.ds(j * LANES, LANES)] = zeros

    for cp in copies:
        cp.wait()

    lane_base = lax.iota(jnp.int32, LANES) * ROW_W

    for k in range(2 * NBLK):
        img = 0 if k < NBLK else 1
        ch6 = img * C + (k % C)
        bidx = lane_base + ch6 * CH_W
        base = k * CHUNK

        @plsc.parallel_loop(0, VECS, unroll=UNROLL)
        def bin_body(i, base=base, bidx=bidx):
            v = buf[pl.ds(base + i * LANES, LANES)]
            y = v * jnp.float32(BINS) + jnp.float32(0.5)
            y = jnp.minimum(jnp.maximum(y, jnp.float32(0.0)),
                            jnp.float32(BINS + 0.49))
            ji = y.astype(jnp.int32)
            f = y - ji.astype(jnp.float32)
            idx0 = bidx + ji
            plsc.addupdate_scatter(acc, [idx0], jnp.float32(1.0) - f)
            plsc.addupdate_scatter(acc, [idx0 + 1], f)

    @plsc.parallel_loop(0, ACC_VECS, unroll=2)
    def red_body(cidx):
        start = cidx * LANES
        s = acc[pl.ds(start, LANES)]
        for r in range(1, LANES):
            s = s + acc[pl.ds(r * ROW_W + start, LANES)]
        part[pl.ds(start, LANES)] = s

    pltpu.sync_copy(part, out_hbm.at[wid])


def _tc_loss(p_ref, o_ref):
    p = p_ref[...]
    h = jnp.sum(p, axis=0)
    slot = lax.broadcasted_iota(jnp.int32, h.shape, 2)
    inb = jnp.logical_and(slot >= 1, slot <= BINS)
    h = jnp.where(inb, h, jnp.float32(0.0))
    tot = jnp.sum(h, axis=2, keepdims=True)
    r = h / (tot + jnp.float32(1e-10))
    r = jnp.where(inb, r + jnp.float32(1e-16), jnp.float32(1e-16))
    a = r[0]
    b = r[1]
    o_ref[0, 0] = jnp.float32(2.0) * jnp.sum((a - b) ** 2 / (a + b))


def kernel(fake_img_s, fake_img_t):
    s_flat = fake_img_s.reshape(-1)
    t_flat = fake_img_t.reshape(-1)

    mesh = plsc.VectorSubcoreMesh(
        core_axis_name="c", subcore_axis_name="s", num_cores=NC, num_subcores=NS
    )
    partials = pl.kernel(
        _sc_hist,
        out_type=jax.ShapeDtypeStruct((NW, ROW_W), jnp.float32),
        mesh=mesh,
        compiler_params=pltpu.CompilerParams(needs_layout_passes=False),
        scratch_types=[
            pltpu.VMEM((2 * NBLK * CHUNK,), jnp.float32),
            pltpu.VMEM((LANES * ROW_W,), jnp.float32),
            pltpu.VMEM((ROW_W,), jnp.float32),
            pltpu.SemaphoreType.DMA,
        ],
    )(s_flat, t_flat)

    loss = pl.pallas_call(
        _tc_loss,
        out_shape=jax.ShapeDtypeStruct((1, 1), jnp.float32),
        out_specs=pl.BlockSpec(memory_space=pltpu.SMEM),
    )(partials.reshape(NW, 2, C, CH_W))
    return loss[0, 0]


# speedup vs baseline: 7.7637x; 1.1361x over previous
import jax
import jax.numpy as jnp
from jax import lax
from jax.experimental import pallas as pl
from jax.experimental.pallas import tpu as pltpu
from jax.experimental.pallas import tpu_sc as plsc

BINS = 256
NC = 2
NS = 16
NW = NC * NS
LANES = 16

B, C, H, W = 4, 3, 224, 224
NBLK = B * C
GRP = H // 8
UPW = NBLK * GRP // (NW // 2)
UPIX = 8 * W
UVECS = UPIX // LANES
WVEC = W // LANES
UNROLL = 7
NCH = 2 * C
CH_W = 272
ROW_W = NCH * CH_W
ACC_VECS = ROW_W // LANES


def _sc_hist(s_hbm, t_hbm, out_hbm, buf, acc, part, sem):
    wid = lax.axis_index("s") * NC + lax.axis_index("c")
    ws = lax.rem(wid, NW // 2)

    def issue(src):
        for ul in range(UPW):
            u = ws * UPW + ul
            kimg = u // GRP
            g = lax.rem(u, GRP)
            pltpu.async_copy(
                src.at[kimg // C, lax.rem(kimg, C), pl.ds(g * 8, 8), :],
                buf.at[ul],
                sem,
            )

    @pl.when(wid < NW // 2)
    def _():
        issue(s_hbm)

    @pl.when(wid >= NW // 2)
    def _():
        issue(t_hbm)

    zeros = jnp.zeros((LANES,), jnp.float32)

    @plsc.parallel_loop(0, LANES * ACC_VECS, unroll=8)
    def zero_body(j):
        acc[pl.ds(j * LANES, LANES)] = zeros

    for ul in range(UPW):
        pltpu.make_async_copy(
            s_hbm.at[0, 0, pl.ds(0, 8), :], buf.at[ul], sem
        ).wait()

    lane_base = lax.iota(jnp.int32, LANES) * ROW_W
    img_off = jnp.where(wid >= NW // 2, C * CH_W, 0)

    def unit_body(ul, _):
        kimg = (ws * UPW + ul) // GRP
        bidx = lane_base + (img_off + lax.rem(kimg, C) * CH_W)

        @plsc.parallel_loop(0, UVECS, unroll=UNROLL)
        def bin_body(i):
            r = i // WVEC
            col = lax.rem(i, WVEC)
            v = buf[ul, r, pl.ds(col * LANES, LANES)]
            y = v * jnp.float32(BINS) + jnp.float32(0.5)
            y = jnp.minimum(jnp.maximum(y, jnp.float32(0.0)),
                            jnp.float32(BINS + 0.49))
            ji = y.astype(jnp.int32)
            f = y - ji.astype(jnp.float32)
            idx0 = bidx + ji
            plsc.addupdate_scatter(acc, [idx0], jnp.float32(1.0) - f)
            plsc.addupdate_scatter(acc, [idx0 + 1], f)

        return _

    lax.fori_loop(0, UPW, unit_body, None)

    @plsc.parallel_loop(0, ACC_VECS, unroll=2)
    def red_body(cidx):
        start = cidx * LANES
        s = acc[pl.ds(start, LANES)]
        for r in range(1, LANES):
            s = s + acc[pl.ds(r * ROW_W + start, LANES)]
        part[pl.ds(start, LANES)] = s

    pltpu.sync_copy(part, out_hbm.at[wid])


def _tc_loss(p_ref, o_ref):
    p = p_ref[...]
    h = jnp.sum(p, axis=0, keepdims=True)
    chans = [h[:, c * CH_W + 1:c * CH_W + 1 + BINS] for c in range(NCH)]
    a = jnp.concatenate(chans[:C], axis=0)
    b = jnp.concatenate(chans[C:], axis=0)
    a = a / (jnp.sum(a, axis=1, keepdims=True) + jnp.float32(1e-10))
    b = b / (jnp.sum(b, axis=1, keepdims=True) + jnp.float32(1e-10))
    a = a + jnp.float32(1e-16)
    b = b + jnp.float32(1e-16)
    o_ref[0, 0] = jnp.float32(2.0) * jnp.sum((a - b) ** 2 / (a + b))


def kernel(fake_img_s, fake_img_t):
    mesh = plsc.VectorSubcoreMesh(
        core_axis_name="c", subcore_axis_name="s", num_cores=NC, num_subcores=NS
    )
    partials = pl.kernel(
        _sc_hist,
        out_type=jax.ShapeDtypeStruct((NW, ROW_W), jnp.float32),
        mesh=mesh,
        compiler_params=pltpu.CompilerParams(needs_layout_passes=False),
        scratch_types=[
            pltpu.VMEM((UPW, 8, W), jnp.float32),
            pltpu.VMEM((LANES * ROW_W,), jnp.float32),
            pltpu.VMEM((ROW_W,), jnp.float32),
            pltpu.SemaphoreType.DMA,
        ],
    )(fake_img_s, fake_img_t)

    loss = pl.pallas_call(
        _tc_loss,
        out_shape=jax.ShapeDtypeStruct((1, 1), jnp.float32),
        out_specs=pl.BlockSpec(memory_space=pltpu.SMEM),
    )(partials)
    return loss[0, 0]


# speedup vs baseline: 8.7412x; 1.1259x over previous
import jax
import jax.numpy as jnp
from jax import lax
from jax.experimental import pallas as pl
from jax.experimental.pallas import tpu as pltpu
from jax.experimental.pallas import tpu_sc as plsc

BINS = 256
NC = 2
NS = 16
NW = NC * NS
LANES = 16

B, C, H, W = 4, 3, 224, 224
NBLK = B * C
GRP = H // 8
UPW = NBLK * GRP // (NW // 2)
UVECS = 8 * W // LANES
UNROLL = 8
NCH = 2 * C
CH_W = 272
ROW_W = NCH * CH_W
ACC_VECS = ROW_W // LANES


def _sc_hist(s_hbm, t_hbm, out_hbm, buf, acc, part, sem):
    wid = lax.axis_index("s") * NC + lax.axis_index("c")
    ws = lax.rem(wid, NW // 2)

    def issue(src):
        for ul in range(UPW):
            u = ws * UPW + ul
            kimg = u // GRP
            g = lax.rem(u, GRP)
            pltpu.async_copy(
                src.at[kimg // C, lax.rem(kimg, C), pl.ds(g * 8, 8), :],
                buf.at[ul],
                sem,
            )

    @pl.when(wid < NW // 2)
    def _():
        issue(s_hbm)

    @pl.when(wid >= NW // 2)
    def _():
        issue(t_hbm)

    zeros = jnp.zeros((LANES,), jnp.float32)

    @plsc.parallel_loop(0, LANES * ACC_VECS, unroll=8)
    def zero_body(j):
        acc[pl.ds(j * LANES, LANES)] = zeros

    for ul in range(UPW):
        pltpu.make_async_copy(
            s_hbm.at[0, 0, pl.ds(0, 8), :], buf.at[ul], sem
        ).wait()

    lane_base = lax.iota(jnp.int32, LANES) * ROW_W
    img_off = jnp.where(wid >= NW // 2, C * CH_W, 0)

    def unit_body(ul, _):
        kimg = (ws * UPW + ul) // GRP
        bidx = lane_base + (img_off + lax.rem(kimg, C) * CH_W)

        @plsc.parallel_loop(0, UVECS, unroll=UNROLL)
        def bin_body(i):
            v = buf[ul, jnp.bitwise_and(i, 7), pl.ds((i >> 3) * LANES, LANES)]
            y = v * jnp.float32(BINS) + jnp.float32(0.5)
            y = jnp.minimum(jnp.maximum(y, jnp.float32(0.0)),
                            jnp.float32(BINS + 0.49))
            ji = y.astype(jnp.int32)
            f = y - ji.astype(jnp.float32)
            idx0 = bidx + ji
            plsc.addupdate_scatter(acc, [idx0], jnp.float32(1.0) - f)
            plsc.addupdate_scatter(acc, [idx0 + 1], f)

        return _

    lax.fori_loop(0, UPW, unit_body, None)

    @plsc.parallel_loop(0, ACC_VECS, unroll=2)
    def red_body(cidx):
        start = cidx * LANES
        s = acc[pl.ds(start, LANES)]
        for r in range(1, LANES):
            s = s + acc[pl.ds(r * ROW_W + start, LANES)]
        part[pl.ds(start, LANES)] = s

    pltpu.sync_copy(part, out_hbm.at[wid])


def _tc_loss(p_ref, o_ref):
    p = p_ref[...]
    h = jnp.sum(p, axis=0, keepdims=True)
    chans = [h[:, c * CH_W + 1:c * CH_W + 1 + BINS] for c in range(NCH)]
    a = jnp.concatenate(chans[:C], axis=0)
    b = jnp.concatenate(chans[C:], axis=0)
    a = a / (jnp.sum(a, axis=1, keepdims=True) + jnp.float32(1e-10))
    b = b / (jnp.sum(b, axis=1, keepdims=True) + jnp.float32(1e-10))
    a = a + jnp.float32(1e-16)
    b = b + jnp.float32(1e-16)
    o_ref[0, 0] = jnp.float32(2.0) * jnp.sum((a - b) ** 2 / (a + b))


def kernel(fake_img_s, fake_img_t):
    mesh = plsc.VectorSubcoreMesh(
        core_axis_name="c", subcore_axis_name="s", num_cores=NC, num_subcores=NS
    )
    partials = pl.kernel(
        _sc_hist,
        out_type=jax.ShapeDtypeStruct((NW, ROW_W), jnp.float32),
        mesh=mesh,
        compiler_params=pltpu.CompilerParams(needs_layout_passes=False),
        scratch_types=[
            pltpu.VMEM((UPW, 8, W), jnp.float32),
            pltpu.VMEM((LANES * ROW_W,), jnp.float32),
            pltpu.VMEM((ROW_W,), jnp.float32),
            pltpu.SemaphoreType.DMA,
        ],
    )(fake_img_s, fake_img_t)

    loss = pl.pallas_call(
        _tc_loss,
        out_shape=jax.ShapeDtypeStruct((1, 1), jnp.float32),
        out_specs=pl.BlockSpec(memory_space=pltpu.SMEM),
    )(partials)
    return loss[0, 0]


# speedup vs baseline: 8.7972x; 1.0064x over previous
import jax
import jax.numpy as jnp
from jax import lax
from jax.experimental import pallas as pl
from jax.experimental.pallas import tpu as pltpu
from jax.experimental.pallas import tpu_sc as plsc

BINS = 256
NC = 2
NS = 16
NW = NC * NS
LANES = 16

B, C, H, W = 4, 3, 224, 224
NBLK = B * C
GRP = H // 8
UPW = NBLK * GRP // (NW // 2)
UVECS = 8 * W // LANES
UNROLL = 8
NCH = 2 * C
CH_W = 272
ROW_W = NCH * CH_W
ACC_VECS = ROW_W // LANES


def _sc_hist(s_hbm, t_hbm, out_hbm, buf, acc, part, sem):
    wid = lax.axis_index("s") * NC + lax.axis_index("c")
    ws = lax.rem(wid, NW // 2)

    def issue(src):
        for ul in range(UPW):
            u = ws * UPW + ul
            kimg = u // GRP
            g = lax.rem(u, GRP)
            pltpu.async_copy(
                src.at[kimg // C, lax.rem(kimg, C), pl.ds(g * 8, 8), :],
                buf.at[ul],
                sem,
            )

    @pl.when(wid < NW // 2)
    def _():
        issue(s_hbm)

    @pl.when(wid >= NW // 2)
    def _():
        issue(t_hbm)

    zeros = jnp.zeros((LANES,), jnp.float32)

    @plsc.parallel_loop(0, LANES * ACC_VECS, unroll=8)
    def zero_body(j):
        acc[pl.ds(j * LANES, LANES)] = zeros

    for ul in range(UPW):
        pltpu.make_async_copy(
            s_hbm.at[0, 0, pl.ds(0, 8), :], buf.at[ul], sem
        ).wait()

    lane_base = lax.iota(jnp.int32, LANES) * ROW_W
    img_off = jnp.where(wid >= NW // 2, C * CH_W, 0)

    def unit_body(ul, _):
        kimg = (ws * UPW + ul) // GRP
        bidx = lane_base + (img_off + lax.rem(kimg, C) * CH_W)

        @plsc.parallel_loop(0, UVECS, unroll=UNROLL)
        def bin_body(i):
            v = buf[ul, jnp.bitwise_and(i, 7), pl.ds((i >> 3) * LANES, LANES)]
            y = v * jnp.float32(BINS) + jnp.float32(0.5)
            ji = y.astype(jnp.int32)
            f = y - ji.astype(jnp.float32)
            idx0 = bidx + ji
            plsc.addupdate_scatter(acc, [idx0], jnp.float32(1.0) - f)
            plsc.addupdate_scatter(acc, [idx0 + 1], f)

        return _

    lax.fori_loop(0, UPW, unit_body, None)

    @plsc.parallel_loop(0, ACC_VECS, unroll=2)
    def red_body(cidx):
        start = cidx * LANES
        s = acc[pl.ds(start, LANES)]
        for r in range(1, LANES):
            s = s + acc[pl.ds(r * ROW_W + start, LANES)]
        part[pl.ds(start, LANES)] = s

    pltpu.sync_copy(part, out_hbm.at[wid])


def _tc_loss(p_ref, o_ref):
    p = p_ref[...]
    h = jnp.sum(p, axis=0, keepdims=True)
    chans = [h[:, c * CH_W + 1:c * CH_W + 1 + BINS] for c in range(NCH)]
    a = jnp.concatenate(chans[:C], axis=0)
    b = jnp.concatenate(chans[C:], axis=0)
    a = a / (jnp.sum(a, axis=1, keepdims=True) + jnp.float32(1e-10))
    b = b / (jnp.sum(b, axis=1, keepdims=True) + jnp.float32(1e-10))
    a = a + jnp.float32(1e-16)
    b = b + jnp.float32(1e-16)
    o_ref[0, 0] = jnp.float32(2.0) * jnp.sum((a - b) ** 2 / (a + b))


def kernel(fake_img_s, fake_img_t):
    mesh = plsc.VectorSubcoreMesh(
        core_axis_name="c", subcore_axis_name="s", num_cores=NC, num_subcores=NS
    )
    partials = pl.kernel(
        _sc_hist,
        out_type=jax.ShapeDtypeStruct((NW, ROW_W), jnp.float32),
        mesh=mesh,
        compiler_params=pltpu.CompilerParams(needs_layout_passes=False),
        scratch_types=[
            pltpu.VMEM((UPW, 8, W), jnp.float32),
            pltpu.VMEM((LANES * ROW_W,), jnp.float32),
            pltpu.VMEM((ROW_W,), jnp.float32),
            pltpu.SemaphoreType.DMA,
        ],
    )(fake_img_s, fake_img_t)

    loss = pl.pallas_call(
        _tc_loss,
        out_shape=jax.ShapeDtypeStruct((1, 1), jnp.float32),
        out_specs=pl.BlockSpec(memory_space=pltpu.SMEM),
    )(partials)
    return loss[0, 0]
